# Initial kernel scaffold; baseline (speedup 1.0000x reference)
import jax
import jax.numpy as jnp
from jax.experimental import pallas as pl


def kernel(x, edge_index, edge_attr, bn_w, bn_b, sage1_wl, sage1_bl, sage1_wr, cbn1_w, cbn1_b, sage2_wl, sage2_bl, sage2_wr, cbn2_w, cbn2_b, sage3_wl, sage3_bl, sage3_wr, cbn3_w, cbn3_b, sage4_wl, sage4_bl, sage4_wr, cbn4_w, cbn4_b, out_w, out_b):
    raise NotImplementedError("write your pallas kernel here")


# speedup vs baseline: 4.0512x; 4.0512x over previous
import functools

import jax
import jax.numpy as jnp
from jax import lax
from jax.experimental import pallas as pl
from jax.experimental.pallas import tpu as pltpu
from jax.experimental.pallas import tpu_sc as plsc

N = 100000
E = 3200000
H = 50000
HP = 51200
DUMP = 50000
B = 128
NB = 25008
NBT = NB // 16
DR = HP // 16
EPAD = NB * B - E
EPS = 1e-5

R = 2000
G = N // R


def _make_segsum(F):
    mesh = plsc.VectorSubcoreMesh(core_axis_name="c", subcore_axis_name="s")

    @functools.partial(
        pl.kernel,
        out_type=jax.ShapeDtypeStruct((2 * HP, F), jnp.float32),
        mesh=mesh,
        compiler_params=pltpu.CompilerParams(use_tc_tiling_on_sc=False),
        scratch_types=[
            pltpu.VMEM((B,), jnp.int32),
            pltpu.VMEM((1, B), jnp.int32),
            pltpu.VMEM((B, F), jnp.float32),
            pltpu.VMEM_SHARED((HP, F), jnp.float32),
            pltpu.SemaphoreType.DMA,
        ],
    )
    def seg(table, src, dstl, zblk, out, src_v, lidx_v, rows_v, acc, sem):
        c = lax.axis_index("c")
        s = lax.axis_index("s")
        pltpu.sync_copy(zblk, acc.at[pl.ds(s * DR, DR)])
        plsc.subcore_barrier()
        tbase = s * NBT

        def body(i, carry):
            blk = tbase + i
            pltpu.sync_copy(src.at[pl.ds(blk * B, B)], src_v)
            pltpu.sync_copy(dstl.at[c, blk], lidx_v)
            pltpu.async_copy(table.at[src_v], rows_v, sem).wait()
            pltpu.sync_copy(rows_v, acc.at[lidx_v.at[0]], add=True)
            return carry

        lax.fori_loop(0, NBT, body, 0)
        plsc.subcore_barrier()
        pltpu.sync_copy(acc.at[pl.ds(s * DR, DR)],
                        out.at[pl.ds(c * HP + s * DR, DR)])

    return seg


_seg8 = _make_segsum(8)
_seg32 = _make_segsum(32)


def _segsum(segk, table, srcp, dstl, zblk):
    agg = segk(table, srcp, dstl, zblk)
    return jnp.concatenate([agg[:H], agg[HP:HP + H]], axis=0)


def _row(F):
    return pl.BlockSpec((R, F), lambda i: (i, 0))


def _full(a, b):
    return pl.BlockSpec((a, b), lambda i: (0, 0))


def _acc_stats(i, y, s_ref, q_ref):
    @pl.when(i == 0)
    def _():
        s_ref[...] = jnp.zeros_like(s_ref)
        q_ref[...] = jnp.zeros_like(q_ref)
    s_ref[...] += jnp.sum(y, axis=0, keepdims=True)
    q_ref[...] += jnp.sum(y * y, axis=0, keepdims=True)


def _stats_x(x4):
    def body(x_ref, s_ref, q_ref):
        _acc_stats(pl.program_id(0), x_ref[...], s_ref, q_ref)
    return pl.pallas_call(
        body, grid=(G,),
        in_specs=[_row(4)],
        out_specs=[_full(1, 4), _full(1, 4)],
        out_shape=[jax.ShapeDtypeStruct((1, 4), jnp.float32)] * 2,
    )(x4)


def _mr(s, q):
    m = s / N
    v = q / N - m * m
    return m, 1.0 / jnp.sqrt(v + EPS)


def _build_t1(x4, m, r, w, b):
    def body(x_ref, m_ref, r_ref, w_ref, b_ref, t_ref):
        h = (x_ref[...] - m_ref[...]) * r_ref[...] * w_ref[...] + b_ref[...]
        t_ref[...] = jnp.concatenate(
            [h, jnp.ones((R, 1), jnp.float32), jnp.zeros((R, 3), jnp.float32)],
            axis=1)
    return pl.pallas_call(
        body, grid=(G,),
        in_specs=[_row(4)] + [_full(1, 4)] * 4,
        out_specs=_row(8),
        out_shape=jax.ShapeDtypeStruct((N, 8), jnp.float32),
    )(x4, m, r, w, b)


def _dot(a, w):
    return jax.lax.dot_general(a, w, (((1,), (0,)), ((), ())),
                               preferred_element_type=jnp.float32)


def _layer1_y(agg, t1, inv, wl8, bl, wr8):
    def body(a_ref, t_ref, i_ref, wl_ref, bl_ref, wr_ref, y_ref, s_ref, q_ref):
        a = a_ref[...] * i_ref[...]
        y = _dot(a, wl_ref[...]) + bl_ref[...] + _dot(t_ref[...], wr_ref[...])
        y_ref[...] = y
        _acc_stats(pl.program_id(0), y, s_ref, q_ref)
    return pl.pallas_call(
        body, grid=(G,),
        in_specs=[_row(8), _row(8), _row(1), _full(8, 64), _full(1, 64),
                  _full(8, 64)],
        out_specs=[_row(64), _full(1, 64), _full(1, 64)],
        out_shape=[jax.ShapeDtypeStruct((N, 64), jnp.float32),
                   jax.ShapeDtypeStruct((1, 64), jnp.float32),
                   jax.ShapeDtypeStruct((1, 64), jnp.float32)],
    )(agg, t1, inv, wl8, bl, wr8)


def _bn_relu_split(y, m, r, w, b):
    def body(y_ref, m_ref, r_ref, w_ref, b_ref, ha_ref, hb_ref):
        h = jax.nn.relu((y_ref[...] - m_ref[...]) * r_ref[...] * w_ref[...]
                        + b_ref[...])
        ha_ref[...] = h[:, :32]
        hb_ref[...] = h[:, 32:]
    return pl.pallas_call(
        body, grid=(G,),
        in_specs=[_row(64)] + [_full(1, 64)] * 4,
        out_specs=[_row(32), _row(32)],
        out_shape=[jax.ShapeDtypeStruct((N, 32), jnp.float32)] * 2,
    )(y, m, r, w, b)


def _layer2_y(aa, ab, ha, hb, inv, wl, bl, wr):
    def body(aa_ref, ab_ref, ha_ref, hb_ref, i_ref, wl_ref, bl_ref, wr_ref,
             y_ref, s_ref, q_ref):
        a = jnp.concatenate([aa_ref[...], ab_ref[...]], axis=1) * i_ref[...]
        h = jnp.concatenate([ha_ref[...], hb_ref[...]], axis=1)
        y = _dot(a, wl_ref[...]) + bl_ref[...] + _dot(h, wr_ref[...])
        y_ref[...] = y
        _acc_stats(pl.program_id(0), y, s_ref, q_ref)
    return pl.pallas_call(
        body, grid=(G,),
        in_specs=[_row(32), _row(32), _row(32), _row(32), _row(1),
                  _full(64, 128), _full(1, 128), _full(64, 128)],
        out_specs=[_row(128), _full(1, 128), _full(1, 128)],
        out_shape=[jax.ShapeDtypeStruct((N, 128), jnp.float32),
                   jax.ShapeDtypeStruct((1, 128), jnp.float32),
                   jax.ShapeDtypeStruct((1, 128), jnp.float32)],
    )(aa, ab, ha, hb, inv, wl, bl, wr)


def _bn_relu_proj2(y, m, r, w, b, wla, wlb):
    def body(y_ref, m_ref, r_ref, w_ref, b_ref, wa_ref, wb_ref,
             h_ref, pa_ref, pb_ref):
        h = jax.nn.relu((y_ref[...] - m_ref[...]) * r_ref[...] * w_ref[...]
                        + b_ref[...])
        h_ref[...] = h
        pa_ref[...] = _dot(h, wa_ref[...])
        pb_ref[...] = _dot(h, wb_ref[...])
    return pl.pallas_call(
        body, grid=(G,),
        in_specs=[_row(128)] + [_full(1, 128)] * 4
                 + [_full(128, 32), _full(128, 32)],
        out_specs=[_row(128), _row(32), _row(32)],
        out_shape=[jax.ShapeDtypeStruct((N, 128), jnp.float32),
                   jax.ShapeDtypeStruct((N, 32), jnp.float32),
                   jax.ShapeDtypeStruct((N, 32), jnp.float32)],
    )(y, m, r, w, b, wla, wlb)


def _layer3_y(pa, pb, h2, inv, bl, wr):
    def body(pa_ref, pb_ref, h_ref, i_ref, bl_ref, wr_ref, y_ref, s_ref, q_ref):
        a = jnp.concatenate([pa_ref[...], pb_ref[...]], axis=1) * i_ref[...]
        y = a + bl_ref[...] + _dot(h_ref[...], wr_ref[...])
        y_ref[...] = y
        _acc_stats(pl.program_id(0), y, s_ref, q_ref)
    return pl.pallas_call(
        body, grid=(G,),
        in_specs=[_row(32), _row(32), _row(128), _row(1), _full(1, 64),
                  _full(128, 64)],
        out_specs=[_row(64), _full(1, 64), _full(1, 64)],
        out_shape=[jax.ShapeDtypeStruct((N, 64), jnp.float32),
                   jax.ShapeDtypeStruct((1, 64), jnp.float32),
                   jax.ShapeDtypeStruct((1, 64), jnp.float32)],
    )(pa, pb, h2, inv, bl, wr)


def _bn_relu_proj1(y, m, r, w, b, wl):
    def body(y_ref, m_ref, r_ref, w_ref, b_ref, wl_ref, h_ref, p_ref):
        h = jax.nn.relu((y_ref[...] - m_ref[...]) * r_ref[...] * w_ref[...]
                        + b_ref[...])
        h_ref[...] = h
        p_ref[...] = _dot(h, wl_ref[...])
    return pl.pallas_call(
        body, grid=(G,),
        in_specs=[_row(64)] + [_full(1, 64)] * 4 + [_full(64, 32)],
        out_specs=[_row(64), _row(32)],
        out_shape=[jax.ShapeDtypeStruct((N, 64), jnp.float32),
                   jax.ShapeDtypeStruct((N, 32), jnp.float32)],
    )(y, m, r, w, b, wl)


def _layer4_y(p4, h3, inv, bl, wr):
    def body(p_ref, h_ref, i_ref, bl_ref, wr_ref, y_ref, s_ref, q_ref):
        y = p_ref[...] * i_ref[...] + bl_ref[...] + _dot(h_ref[...], wr_ref[...])
        y_ref[...] = y
        _acc_stats(pl.program_id(0), y, s_ref, q_ref)
    return pl.pallas_call(
        body, grid=(G,),
        in_specs=[_row(32), _row(64), _row(1), _full(1, 32), _full(64, 32)],
        out_specs=[_row(32), _full(1, 32), _full(1, 32)],
        out_shape=[jax.ShapeDtypeStruct((N, 32), jnp.float32),
                   jax.ShapeDtypeStruct((1, 32), jnp.float32),
                   jax.ShapeDtypeStruct((1, 32), jnp.float32)],
    )(p4, h3, inv, bl, wr)


def _head(y, m, r, w, b, ow, ob):
    def body(y_ref, m_ref, r_ref, w_ref, b_ref, ow_ref, ob_ref, o_ref):
        h = jax.nn.relu((y_ref[...] - m_ref[...]) * r_ref[...] * w_ref[...]
                        + b_ref[...])
        o_ref[...] = _dot(h, ow_ref[...]) + ob_ref[...]
    return pl.pallas_call(
        body, grid=(G,),
        in_specs=[_row(32)] + [_full(1, 32)] * 4 + [_full(32, 1), _full(1, 1)],
        out_specs=_row(1),
        out_shape=jax.ShapeDtypeStruct((N, 1), jnp.float32),
    )(y, m, r, w, b, ow, ob)


def _v(a):
    return a.reshape(1, -1)


def kernel(x, edge_index, edge_attr, bn_w, bn_b,
           sage1_wl, sage1_bl, sage1_wr, cbn1_w, cbn1_b,
           sage2_wl, sage2_bl, sage2_wr, cbn2_w, cbn2_b,
           sage3_wl, sage3_bl, sage3_wr, cbn3_w, cbn3_b,
           sage4_wl, sage4_bl, sage4_wr, cbn4_w, cbn4_b,
           out_w, out_b):
    src, dst = edge_index[0], edge_index[1]
    srcp = jnp.concatenate([src, jnp.zeros((EPAD,), jnp.int32)])
    loc = dst[None, :] - jnp.array([[0], [H]], jnp.int32)
    locc = jnp.where((loc >= 0) & (loc < H), loc, DUMP)
    dstl = jnp.concatenate(
        [locc, jnp.full((2, EPAD), DUMP, jnp.int32)], axis=1
    ).reshape(2, NB, 1, B)
    z8 = jnp.zeros((DR, 8), jnp.float32)
    z32 = jnp.zeros((DR, 32), jnp.float32)

    x4 = x[:, 0:4]
    s0, q0 = _stats_x(x4)
    m0, r0 = _mr(s0, q0)
    t1 = _build_t1(x4, m0, r0, _v(bn_w), _v(bn_b))

    agg1 = _segsum(_seg8, t1, srcp, dstl, z8)
    inv = (1.0 / jnp.clip(agg1[:, 4], 1.0, None)).reshape(N, 1)
    wl8 = jnp.zeros((8, 64), jnp.float32).at[:4].set(sage1_wl)
    wr8 = jnp.zeros((8, 64), jnp.float32).at[:4].set(sage1_wr)
    y1, s1, q1 = _layer1_y(agg1, t1, inv, wl8, _v(sage1_bl), wr8)
    m1, r1 = _mr(s1, q1)
    h1a, h1b = _bn_relu_split(y1, m1, r1, _v(cbn1_w), _v(cbn1_b))

    a2a = _segsum(_seg32, h1a, srcp, dstl, z32)
    a2b = _segsum(_seg32, h1b, srcp, dstl, z32)
    y2, s2, q2 = _layer2_y(a2a, a2b, h1a, h1b, inv,
                           sage2_wl, _v(sage2_bl), sage2_wr)
    m2, r2 = _mr(s2, q2)
    h2, p3a, p3b = _bn_relu_proj2(y2, m2, r2, _v(cbn2_w), _v(cbn2_b),
                                  sage3_wl[:, :32], sage3_wl[:, 32:])

    P3a = _segsum(_seg32, p3a, srcp, dstl, z32)
    P3b = _segsum(_seg32, p3b, srcp, dstl, z32)
    y3, s3, q3 = _layer3_y(P3a, P3b, h2, inv, _v(sage3_bl), sage3_wr)
    m3, r3 = _mr(s3, q3)
    h3, p4 = _bn_relu_proj1(y3, m3, r3, _v(cbn3_w), _v(cbn3_b), sage4_wl)

    P4 = _segsum(_seg32, p4, srcp, dstl, z32)
    y4, s4, q4 = _layer4_y(P4, h3, inv, _v(sage4_bl), sage4_wr)
    m4, r4 = _mr(s4, q4)
    out = _head(y4, m4, r4, _v(cbn4_w), _v(cbn4_b), out_w, _v(out_b))
    return out[:, 0]


# speedup vs baseline: 5.9713x; 1.4740x over previous
import functools

import jax
import jax.numpy as jnp
from jax import lax
from jax.experimental import pallas as pl
from jax.experimental.pallas import tpu as pltpu
from jax.experimental.pallas import tpu_sc as plsc

N = 100000
E = 3200000
H = 50000
HP = 51200
DUMP = 50000
B = 128
NB = 25024
NBT = NB // 16
DR = HP // 16
EPAD = NB * B - E
EPS = 1e-5

R = 2000
G = N // R


def _make_segsum(F):
    mesh = plsc.VectorSubcoreMesh(core_axis_name="c", subcore_axis_name="s")

    @functools.partial(
        pl.kernel,
        out_type=jax.ShapeDtypeStruct((2 * HP, F), jnp.float32),
        mesh=mesh,
        compiler_params=pltpu.CompilerParams(use_tc_tiling_on_sc=False),
        scratch_types=[
            pltpu.VMEM((B,), jnp.int32),
            pltpu.VMEM((B,), jnp.int32),
            pltpu.VMEM((1, B), jnp.int32),
            pltpu.VMEM((1, B), jnp.int32),
            pltpu.VMEM((B, F), jnp.float32),
            pltpu.VMEM((B, F), jnp.float32),
            pltpu.VMEM_SHARED((HP, F), jnp.float32),
            pltpu.SemaphoreType.DMA,
            pltpu.SemaphoreType.DMA,
        ],
    )
    def seg(table, src, dstl, zblk, out,
            s0v, s1v, l0v, l1v, r0v, r1v, acc, sem0, sem1):
        c = lax.axis_index("c")
        s = lax.axis_index("s")
        pltpu.sync_copy(zblk, acc.at[pl.ds(s * DR, DR)])
        plsc.subcore_barrier()
        tbase = s * NBT
        half = NBT // 2
        pltpu.sync_copy(src.at[pl.ds(tbase * B, B)], s0v)
        pltpu.sync_copy(dstl.at[c, tbase], l0v)
        pltpu.async_copy(table.at[s0v], r0v, sem0)

        def body(i, carry):
            b1 = tbase + 2 * i + 1
            pltpu.sync_copy(src.at[pl.ds(b1 * B, B)], s1v)
            pltpu.sync_copy(dstl.at[c, b1], l1v)
            pltpu.async_copy(table.at[s1v], r1v, sem1)
            pltpu.make_async_copy(table.at[s0v], r0v, sem0).wait()
            pltpu.sync_copy(r0v, acc.at[l0v.at[0]], add=True)

            @pl.when(i < half - 1)
            def _():
                b0 = tbase + 2 * i + 2
                pltpu.sync_copy(src.at[pl.ds(b0 * B, B)], s0v)
                pltpu.sync_copy(dstl.at[c, b0], l0v)
                pltpu.async_copy(table.at[s0v], r0v, sem0)

            pltpu.make_async_copy(table.at[s1v], r1v, sem1).wait()
            pltpu.sync_copy(r1v, acc.at[l1v.at[0]], add=True)
            return carry

        lax.fori_loop(0, half, body, 0)
        plsc.subcore_barrier()
        pltpu.sync_copy(acc.at[pl.ds(s * DR, DR)],
                        out.at[pl.ds(c * HP + s * DR, DR)])

    return seg


_seg8 = _make_segsum(8)
_seg32 = _make_segsum(32)


def _segsum(segk, table, srcp, dstl, zblk):
    agg = segk(table, srcp, dstl, zblk)
    return jnp.concatenate([agg[:H], agg[HP:HP + H]], axis=0)


def _row(F):
    return pl.BlockSpec((R, F), lambda i: (i, 0))


def _full(a, b):
    return pl.BlockSpec((a, b), lambda i: (0, 0))


def _acc_stats(i, y, s_ref, q_ref):
    @pl.when(i == 0)
    def _():
        s_ref[...] = jnp.zeros_like(s_ref)
        q_ref[...] = jnp.zeros_like(q_ref)
    s_ref[...] += jnp.sum(y, axis=0, keepdims=True)
    q_ref[...] += jnp.sum(y * y, axis=0, keepdims=True)


def _stats_x(x4):
    def body(x_ref, s_ref, q_ref):
        _acc_stats(pl.program_id(0), x_ref[...], s_ref, q_ref)
    return pl.pallas_call(
        body, grid=(G,),
        in_specs=[_row(4)],
        out_specs=[_full(1, 4), _full(1, 4)],
        out_shape=[jax.ShapeDtypeStruct((1, 4), jnp.float32)] * 2,
    )(x4)


def _mr(s, q):
    m = s / N
    v = q / N - m * m
    return m, 1.0 / jnp.sqrt(v + EPS)


def _build_t1(x4, m, r, w, b):
    def body(x_ref, m_ref, r_ref, w_ref, b_ref, t_ref):
        h = (x_ref[...] - m_ref[...]) * r_ref[...] * w_ref[...] + b_ref[...]
        t_ref[...] = jnp.concatenate(
            [h, jnp.ones((R, 1), jnp.float32), jnp.zeros((R, 3), jnp.float32)],
            axis=1)
    return pl.pallas_call(
        body, grid=(G,),
        in_specs=[_row(4)] + [_full(1, 4)] * 4,
        out_specs=_row(8),
        out_shape=jax.ShapeDtypeStruct((N, 8), jnp.float32),
    )(x4, m, r, w, b)


def _dot(a, w):
    return jax.lax.dot_general(a, w, (((1,), (0,)), ((), ())),
                               preferred_element_type=jnp.float32)


def _layer1_y(agg, t1, inv, wl8, bl, wr8):
    def body(a_ref, t_ref, i_ref, wl_ref, bl_ref, wr_ref, y_ref, s_ref, q_ref):
        a = a_ref[...] * i_ref[...]
        y = _dot(a, wl_ref[...]) + bl_ref[...] + _dot(t_ref[...], wr_ref[...])
        y_ref[...] = y
        _acc_stats(pl.program_id(0), y, s_ref, q_ref)
    return pl.pallas_call(
        body, grid=(G,),
        in_specs=[_row(8), _row(8), _row(1), _full(8, 64), _full(1, 64),
                  _full(8, 64)],
        out_specs=[_row(64), _full(1, 64), _full(1, 64)],
        out_shape=[jax.ShapeDtypeStruct((N, 64), jnp.float32),
                   jax.ShapeDtypeStruct((1, 64), jnp.float32),
                   jax.ShapeDtypeStruct((1, 64), jnp.float32)],
    )(agg, t1, inv, wl8, bl, wr8)


def _bn_relu_split(y, m, r, w, b):
    def body(y_ref, m_ref, r_ref, w_ref, b_ref, ha_ref, hb_ref):
        h = jax.nn.relu((y_ref[...] - m_ref[...]) * r_ref[...] * w_ref[...]
                        + b_ref[...])
        ha_ref[...] = h[:, :32]
        hb_ref[...] = h[:, 32:]
    return pl.pallas_call(
        body, grid=(G,),
        in_specs=[_row(64)] + [_full(1, 64)] * 4,
        out_specs=[_row(32), _row(32)],
        out_shape=[jax.ShapeDtypeStruct((N, 32), jnp.float32)] * 2,
    )(y, m, r, w, b)


def _layer2_y(aa, ab, ha, hb, inv, wl, bl, wr):
    def body(aa_ref, ab_ref, ha_ref, hb_ref, i_ref, wl_ref, bl_ref, wr_ref,
             y_ref, s_ref, q_ref):
        a = jnp.concatenate([aa_ref[...], ab_ref[...]], axis=1) * i_ref[...]
        h = jnp.concatenate([ha_ref[...], hb_ref[...]], axis=1)
        y = _dot(a, wl_ref[...]) + bl_ref[...] + _dot(h, wr_ref[...])
        y_ref[...] = y
        _acc_stats(pl.program_id(0), y, s_ref, q_ref)
    return pl.pallas_call(
        body, grid=(G,),
        in_specs=[_row(32), _row(32), _row(32), _row(32), _row(1),
                  _full(64, 128), _full(1, 128), _full(64, 128)],
        out_specs=[_row(128), _full(1, 128), _full(1, 128)],
        out_shape=[jax.ShapeDtypeStruct((N, 128), jnp.float32),
                   jax.ShapeDtypeStruct((1, 128), jnp.float32),
                   jax.ShapeDtypeStruct((1, 128), jnp.float32)],
    )(aa, ab, ha, hb, inv, wl, bl, wr)


def _bn_relu_proj2(y, m, r, w, b, wla, wlb):
    def body(y_ref, m_ref, r_ref, w_ref, b_ref, wa_ref, wb_ref,
             h_ref, pa_ref, pb_ref):
        h = jax.nn.relu((y_ref[...] - m_ref[...]) * r_ref[...] * w_ref[...]
                        + b_ref[...])
        h_ref[...] = h
        pa_ref[...] = _dot(h, wa_ref[...])
        pb_ref[...] = _dot(h, wb_ref[...])
    return pl.pallas_call(
        body, grid=(G,),
        in_specs=[_row(128)] + [_full(1, 128)] * 4
                 + [_full(128, 32), _full(128, 32)],
        out_specs=[_row(128), _row(32), _row(32)],
        out_shape=[jax.ShapeDtypeStruct((N, 128), jnp.float32),
                   jax.ShapeDtypeStruct((N, 32), jnp.float32),
                   jax.ShapeDtypeStruct((N, 32), jnp.float32)],
    )(y, m, r, w, b, wla, wlb)


def _layer3_y(pa, pb, h2, inv, bl, wr):
    def body(pa_ref, pb_ref, h_ref, i_ref, bl_ref, wr_ref, y_ref, s_ref, q_ref):
        a = jnp.concatenate([pa_ref[...], pb_ref[...]], axis=1) * i_ref[...]
        y = a + bl_ref[...] + _dot(h_ref[...], wr_ref[...])
        y_ref[...] = y
        _acc_stats(pl.program_id(0), y, s_ref, q_ref)
    return pl.pallas_call(
        body, grid=(G,),
        in_specs=[_row(32), _row(32), _row(128), _row(1), _full(1, 64),
                  _full(128, 64)],
        out_specs=[_row(64), _full(1, 64), _full(1, 64)],
        out_shape=[jax.ShapeDtypeStruct((N, 64), jnp.float32),
                   jax.ShapeDtypeStruct((1, 64), jnp.float32),
                   jax.ShapeDtypeStruct((1, 64), jnp.float32)],
    )(pa, pb, h2, inv, bl, wr)


def _bn_relu_proj1(y, m, r, w, b, wl):
    def body(y_ref, m_ref, r_ref, w_ref, b_ref, wl_ref, h_ref, p_ref):
        h = jax.nn.relu((y_ref[...] - m_ref[...]) * r_ref[...] * w_ref[...]
                        + b_ref[...])
        h_ref[...] = h
        p_ref[...] = _dot(h, wl_ref[...])
    return pl.pallas_call(
        body, grid=(G,),
        in_specs=[_row(64)] + [_full(1, 64)] * 4 + [_full(64, 32)],
        out_specs=[_row(64), _row(32)],
        out_shape=[jax.ShapeDtypeStruct((N, 64), jnp.float32),
                   jax.ShapeDtypeStruct((N, 32), jnp.float32)],
    )(y, m, r, w, b, wl)


def _layer4_y(p4, h3, inv, bl, wr):
    def body(p_ref, h_ref, i_ref, bl_ref, wr_ref, y_ref, s_ref, q_ref):
        y = p_ref[...] * i_ref[...] + bl_ref[...] + _dot(h_ref[...], wr_ref[...])
        y_ref[...] = y
        _acc_stats(pl.program_id(0), y, s_ref, q_ref)
    return pl.pallas_call(
        body, grid=(G,),
        in_specs=[_row(32), _row(64), _row(1), _full(1, 32), _full(64, 32)],
        out_specs=[_row(32), _full(1, 32), _full(1, 32)],
        out_shape=[jax.ShapeDtypeStruct((N, 32), jnp.float32),
                   jax.ShapeDtypeStruct((1, 32), jnp.float32),
                   jax.ShapeDtypeStruct((1, 32), jnp.float32)],
    )(p4, h3, inv, bl, wr)


def _head(y, m, r, w, b, ow, ob):
    def body(y_ref, m_ref, r_ref, w_ref, b_ref, ow_ref, ob_ref, o_ref):
        h = jax.nn.relu((y_ref[...] - m_ref[...]) * r_ref[...] * w_ref[...]
                        + b_ref[...])
        o_ref[...] = _dot(h, ow_ref[...]) + ob_ref[...]
    return pl.pallas_call(
        body, grid=(G,),
        in_specs=[_row(32)] + [_full(1, 32)] * 4 + [_full(32, 1), _full(1, 1)],
        out_specs=_row(1),
        out_shape=jax.ShapeDtypeStruct((N, 1), jnp.float32),
    )(y, m, r, w, b, ow, ob)


def _v(a):
    return a.reshape(1, -1)


def kernel(x, edge_index, edge_attr, bn_w, bn_b,
           sage1_wl, sage1_bl, sage1_wr, cbn1_w, cbn1_b,
           sage2_wl, sage2_bl, sage2_wr, cbn2_w, cbn2_b,
           sage3_wl, sage3_bl, sage3_wr, cbn3_w, cbn3_b,
           sage4_wl, sage4_bl, sage4_wr, cbn4_w, cbn4_b,
           out_w, out_b):
    src, dst = edge_index[0], edge_index[1]
    srcp = jnp.concatenate([src, jnp.zeros((EPAD,), jnp.int32)])
    loc = dst[None, :] - jnp.array([[0], [H]], jnp.int32)
    locc = jnp.where((loc >= 0) & (loc < H), loc, DUMP)
    dstl = jnp.concatenate(
        [locc, jnp.full((2, EPAD), DUMP, jnp.int32)], axis=1
    ).reshape(2, NB, 1, B)
    z8 = jnp.zeros((DR, 8), jnp.float32)
    z32 = jnp.zeros((DR, 32), jnp.float32)

    x4 = x[:, 0:4]
    s0, q0 = _stats_x(x4)
    m0, r0 = _mr(s0, q0)
    t1 = _build_t1(x4, m0, r0, _v(bn_w), _v(bn_b))

    agg1 = _segsum(_seg8, t1, srcp, dstl, z8)
    inv = (1.0 / jnp.clip(agg1[:, 4], 1.0, None)).reshape(N, 1)
    wl8 = jnp.zeros((8, 64), jnp.float32).at[:4].set(sage1_wl)
    wr8 = jnp.zeros((8, 64), jnp.float32).at[:4].set(sage1_wr)
    y1, s1, q1 = _layer1_y(agg1, t1, inv, wl8, _v(sage1_bl), wr8)
    m1, r1 = _mr(s1, q1)
    h1a, h1b = _bn_relu_split(y1, m1, r1, _v(cbn1_w), _v(cbn1_b))

    a2a = _segsum(_seg32, h1a, srcp, dstl, z32)
    a2b = _segsum(_seg32, h1b, srcp, dstl, z32)
    y2, s2, q2 = _layer2_y(a2a, a2b, h1a, h1b, inv,
                           sage2_wl, _v(sage2_bl), sage2_wr)
    m2, r2 = _mr(s2, q2)
    h2, p3a, p3b = _bn_relu_proj2(y2, m2, r2, _v(cbn2_w), _v(cbn2_b),
                                  sage3_wl[:, :32], sage3_wl[:, 32:])

    P3a = _segsum(_seg32, p3a, srcp, dstl, z32)
    P3b = _segsum(_seg32, p3b, srcp, dstl, z32)
    y3, s3, q3 = _layer3_y(P3a, P3b, h2, inv, _v(sage3_bl), sage3_wr)
    m3, r3 = _mr(s3, q3)
    h3, p4 = _bn_relu_proj1(y3, m3, r3, _v(cbn3_w), _v(cbn3_b), sage4_wl)

    P4 = _segsum(_seg32, p4, srcp, dstl, z32)
    y4, s4, q4 = _layer4_y(P4, h3, inv, _v(sage4_bl), sage4_wr)
    m4, r4 = _mr(s4, q4)
    out = _head(y4, m4, r4, _v(cbn4_w), _v(cbn4_b), out_w, _v(out_b))
    return out[:, 0]


# speedup vs baseline: 5.9892x; 1.0030x over previous
import functools

import jax
import jax.numpy as jnp
from jax import lax
from jax.experimental import pallas as pl
from jax.experimental.pallas import tpu as pltpu
from jax.experimental.pallas import tpu_sc as plsc

N = 100000
E = 3200000
H = 50000
HP = 51200
DUMP = 50000
B = 128
NB = 25024
NBT = NB // 16
DR = HP // 16
EPAD = NB * B - E
NBUF = 4
EPS = 1e-5

R = 2000
G = N // R


def _make_segsum(F):
    mesh = plsc.VectorSubcoreMesh(core_axis_name="c", subcore_axis_name="s")

    @functools.partial(
        pl.kernel,
        out_type=jax.ShapeDtypeStruct((2 * HP, F), jnp.float32),
        mesh=mesh,
        compiler_params=pltpu.CompilerParams(use_tc_tiling_on_sc=False),
        scratch_types=(
            [pltpu.VMEM((B,), jnp.int32) for _ in range(NBUF)]
            + [pltpu.VMEM((1, B), jnp.int32) for _ in range(NBUF)]
            + [pltpu.VMEM((B, F), jnp.float32) for _ in range(NBUF)]
            + [pltpu.VMEM_SHARED((HP, F), jnp.float32)]
            + [pltpu.SemaphoreType.DMA for _ in range(NBUF)]
        ),
    )
    def seg(table, src, dstl, zblk, out, *scr):
        sv = scr[0:NBUF]
        lv = scr[NBUF:2 * NBUF]
        rv = scr[2 * NBUF:3 * NBUF]
        acc = scr[3 * NBUF]
        sems = scr[3 * NBUF + 1:]
        c = lax.axis_index("c")
        s = lax.axis_index("s")
        pltpu.sync_copy(zblk, acc.at[pl.ds(s * DR, DR)])
        plsc.subcore_barrier()
        tbase = s * NBT
        ngrp = NBT // NBUF

        def issue(b, blk):
            pltpu.sync_copy(src.at[pl.ds(blk * B, B)], sv[b])
            pltpu.sync_copy(dstl.at[c, blk], lv[b])
            pltpu.async_copy(table.at[sv[b]], rv[b], sems[b])

        for b in range(NBUF):
            issue(b, tbase + b)

        def body(g, carry):
            for b in range(NBUF):
                pltpu.make_async_copy(table.at[sv[b]], rv[b], sems[b]).wait()
                pltpu.sync_copy(rv[b], acc.at[lv[b].at[0]], add=True)

                @pl.when(g < ngrp - 1)
                def _():
                    issue(b, tbase + g * NBUF + b + NBUF)
            return carry

        lax.fori_loop(0, ngrp, body, 0)
        plsc.subcore_barrier()
        pltpu.sync_copy(acc.at[pl.ds(s * DR, DR)],
                        out.at[pl.ds(c * HP + s * DR, DR)])

    return seg


_seg8 = _make_segsum(8)
_seg32 = _make_segsum(32)


def _segsum(segk, table, srcp, dstl, zblk):
    agg = segk(table, srcp, dstl, zblk)
    return jnp.concatenate([agg[:H], agg[HP:HP + H]], axis=0)


def _row(F):
    return pl.BlockSpec((R, F), lambda i: (i, 0))


def _full(a, b):
    return pl.BlockSpec((a, b), lambda i: (0, 0))


def _acc_stats(i, y, s_ref, q_ref):
    @pl.when(i == 0)
    def _():
        s_ref[...] = jnp.zeros_like(s_ref)
        q_ref[...] = jnp.zeros_like(q_ref)
    s_ref[...] += jnp.sum(y, axis=0, keepdims=True)
    q_ref[...] += jnp.sum(y * y, axis=0, keepdims=True)


def _stats_x(x4):
    def body(x_ref, s_ref, q_ref):
        _acc_stats(pl.program_id(0), x_ref[...], s_ref, q_ref)
    return pl.pallas_call(
        body, grid=(G,),
        in_specs=[_row(4)],
        out_specs=[_full(1, 4), _full(1, 4)],
        out_shape=[jax.ShapeDtypeStruct((1, 4), jnp.float32)] * 2,
    )(x4)


def _mr(s, q):
    m = s / N
    v = q / N - m * m
    return m, 1.0 / jnp.sqrt(v + EPS)


def _build_t1(x4, m, r, w, b):
    def body(x_ref, m_ref, r_ref, w_ref, b_ref, t_ref):
        h = (x_ref[...] - m_ref[...]) * r_ref[...] * w_ref[...] + b_ref[...]
        t_ref[...] = jnp.concatenate(
            [h, jnp.ones((R, 1), jnp.float32), jnp.zeros((R, 3), jnp.float32)],
            axis=1)
    return pl.pallas_call(
        body, grid=(G,),
        in_specs=[_row(4)] + [_full(1, 4)] * 4,
        out_specs=_row(8),
        out_shape=jax.ShapeDtypeStruct((N, 8), jnp.float32),
    )(x4, m, r, w, b)


def _dot(a, w):
    return jax.lax.dot_general(a, w, (((1,), (0,)), ((), ())),
                               preferred_element_type=jnp.float32)


def _layer1_y(agg, t1, inv, wl8, bl, wr8):
    def body(a_ref, t_ref, i_ref, wl_ref, bl_ref, wr_ref, y_ref, s_ref, q_ref):
        a = a_ref[...] * i_ref[...]
        y = _dot(a, wl_ref[...]) + bl_ref[...] + _dot(t_ref[...], wr_ref[...])
        y_ref[...] = y
        _acc_stats(pl.program_id(0), y, s_ref, q_ref)
    return pl.pallas_call(
        body, grid=(G,),
        in_specs=[_row(8), _row(8), _row(1), _full(8, 64), _full(1, 64),
                  _full(8, 64)],
        out_specs=[_row(64), _full(1, 64), _full(1, 64)],
        out_shape=[jax.ShapeDtypeStruct((N, 64), jnp.float32),
                   jax.ShapeDtypeStruct((1, 64), jnp.float32),
                   jax.ShapeDtypeStruct((1, 64), jnp.float32)],
    )(agg, t1, inv, wl8, bl, wr8)


def _bn_relu_split(y, m, r, w, b):
    def body(y_ref, m_ref, r_ref, w_ref, b_ref, ha_ref, hb_ref):
        h = jax.nn.relu((y_ref[...] - m_ref[...]) * r_ref[...] * w_ref[...]
                        + b_ref[...])
        ha_ref[...] = h[:, :32]
        hb_ref[...] = h[:, 32:]
    return pl.pallas_call(
        body, grid=(G,),
        in_specs=[_row(64)] + [_full(1, 64)] * 4,
        out_specs=[_row(32), _row(32)],
        out_shape=[jax.ShapeDtypeStruct((N, 32), jnp.float32)] * 2,
    )(y, m, r, w, b)


def _layer2_y(aa, ab, ha, hb, inv, wl, bl, wr):
    def body(aa_ref, ab_ref, ha_ref, hb_ref, i_ref, wl_ref, bl_ref, wr_ref,
             y_ref, s_ref, q_ref):
        a = jnp.concatenate([aa_ref[...], ab_ref[...]], axis=1) * i_ref[...]
        h = jnp.concatenate([ha_ref[...], hb_ref[...]], axis=1)
        y = _dot(a, wl_ref[...]) + bl_ref[...] + _dot(h, wr_ref[...])
        y_ref[...] = y
        _acc_stats(pl.program_id(0), y, s_ref, q_ref)
    return pl.pallas_call(
        body, grid=(G,),
        in_specs=[_row(32), _row(32), _row(32), _row(32), _row(1),
                  _full(64, 128), _full(1, 128), _full(64, 128)],
        out_specs=[_row(128), _full(1, 128), _full(1, 128)],
        out_shape=[jax.ShapeDtypeStruct((N, 128), jnp.float32),
                   jax.ShapeDtypeStruct((1, 128), jnp.float32),
                   jax.ShapeDtypeStruct((1, 128), jnp.float32)],
    )(aa, ab, ha, hb, inv, wl, bl, wr)


def _bn_relu_proj2(y, m, r, w, b, wla, wlb):
    def body(y_ref, m_ref, r_ref, w_ref, b_ref, wa_ref, wb_ref,
             h_ref, pa_ref, pb_ref):
        h = jax.nn.relu((y_ref[...] - m_ref[...]) * r_ref[...] * w_ref[...]
                        + b_ref[...])
        h_ref[...] = h
        pa_ref[...] = _dot(h, wa_ref[...])
        pb_ref[...] = _dot(h, wb_ref[...])
    return pl.pallas_call(
        body, grid=(G,),
        in_specs=[_row(128)] + [_full(1, 128)] * 4
                 + [_full(128, 32), _full(128, 32)],
        out_specs=[_row(128), _row(32), _row(32)],
        out_shape=[jax.ShapeDtypeStruct((N, 128), jnp.float32),
                   jax.ShapeDtypeStruct((N, 32), jnp.float32),
                   jax.ShapeDtypeStruct((N, 32), jnp.float32)],
    )(y, m, r, w, b, wla, wlb)


def _layer3_y(pa, pb, h2, inv, bl, wr):
    def body(pa_ref, pb_ref, h_ref, i_ref, bl_ref, wr_ref, y_ref, s_ref, q_ref):
        a = jnp.concatenate([pa_ref[...], pb_ref[...]], axis=1) * i_ref[...]
        y = a + bl_ref[...] + _dot(h_ref[...], wr_ref[...])
        y_ref[...] = y
        _acc_stats(pl.program_id(0), y, s_ref, q_ref)
    return pl.pallas_call(
        body, grid=(G,),
        in_specs=[_row(32), _row(32), _row(128), _row(1), _full(1, 64),
                  _full(128, 64)],
        out_specs=[_row(64), _full(1, 64), _full(1, 64)],
        out_shape=[jax.ShapeDtypeStruct((N, 64), jnp.float32),
                   jax.ShapeDtypeStruct((1, 64), jnp.float32),
                   jax.ShapeDtypeStruct((1, 64), jnp.float32)],
    )(pa, pb, h2, inv, bl, wr)


def _bn_relu_proj1(y, m, r, w, b, wl):
    def body(y_ref, m_ref, r_ref, w_ref, b_ref, wl_ref, h_ref, p_ref):
        h = jax.nn.relu((y_ref[...] - m_ref[...]) * r_ref[...] * w_ref[...]
                        + b_ref[...])
        h_ref[...] = h
        p_ref[...] = _dot(h, wl_ref[...])
    return pl.pallas_call(
        body, grid=(G,),
        in_specs=[_row(64)] + [_full(1, 64)] * 4 + [_full(64, 32)],
        out_specs=[_row(64), _row(32)],
        out_shape=[jax.ShapeDtypeStruct((N, 64), jnp.float32),
                   jax.ShapeDtypeStruct((N, 32), jnp.float32)],
    )(y, m, r, w, b, wl)


def _layer4_y(p4, h3, inv, bl, wr):
    def body(p_ref, h_ref, i_ref, bl_ref, wr_ref, y_ref, s_ref, q_ref):
        y = p_ref[...] * i_ref[...] + bl_ref[...] + _dot(h_ref[...], wr_ref[...])
        y_ref[...] = y
        _acc_stats(pl.program_id(0), y, s_ref, q_ref)
    return pl.pallas_call(
        body, grid=(G,),
        in_specs=[_row(32), _row(64), _row(1), _full(1, 32), _full(64, 32)],
        out_specs=[_row(32), _full(1, 32), _full(1, 32)],
        out_shape=[jax.ShapeDtypeStruct((N, 32), jnp.float32),
                   jax.ShapeDtypeStruct((1, 32), jnp.float32),
                   jax.ShapeDtypeStruct((1, 32), jnp.float32)],
    )(p4, h3, inv, bl, wr)


def _head(y, m, r, w, b, ow, ob):
    def body(y_ref, m_ref, r_ref, w_ref, b_ref, ow_ref, ob_ref, o_ref):
        h = jax.nn.relu((y_ref[...] - m_ref[...]) * r_ref[...] * w_ref[...]
                        + b_ref[...])
        o_ref[...] = _dot(h, ow_ref[...]) + ob_ref[...]
    return pl.pallas_call(
        body, grid=(G,),
        in_specs=[_row(32)] + [_full(1, 32)] * 4 + [_full(32, 1), _full(1, 1)],
        out_specs=_row(1),
        out_shape=jax.ShapeDtypeStruct((N, 1), jnp.float32),
    )(y, m, r, w, b, ow, ob)


def _v(a):
    return a.reshape(1, -1)


def kernel(x, edge_index, edge_attr, bn_w, bn_b,
           sage1_wl, sage1_bl, sage1_wr, cbn1_w, cbn1_b,
           sage2_wl, sage2_bl, sage2_wr, cbn2_w, cbn2_b,
           sage3_wl, sage3_bl, sage3_wr, cbn3_w, cbn3_b,
           sage4_wl, sage4_bl, sage4_wr, cbn4_w, cbn4_b,
           out_w, out_b):
    src, dst = edge_index[0], edge_index[1]
    srcp = jnp.concatenate([src, jnp.zeros((EPAD,), jnp.int32)])
    loc = dst[None, :] - jnp.array([[0], [H]], jnp.int32)
    locc = jnp.where((loc >= 0) & (loc < H), loc, DUMP)
    dstl = jnp.concatenate(
        [locc, jnp.full((2, EPAD), DUMP, jnp.int32)], axis=1
    ).reshape(2, NB, 1, B)
    z8 = jnp.zeros((DR, 8), jnp.float32)
    z32 = jnp.zeros((DR, 32), jnp.float32)

    x4 = x[:, 0:4]
    s0, q0 = _stats_x(x4)
    m0, r0 = _mr(s0, q0)
    t1 = _build_t1(x4, m0, r0, _v(bn_w), _v(bn_b))

    agg1 = _segsum(_seg8, t1, srcp, dstl, z8)
    inv = (1.0 / jnp.clip(agg1[:, 4], 1.0, None)).reshape(N, 1)
    wl8 = jnp.zeros((8, 64), jnp.float32).at[:4].set(sage1_wl)
    wr8 = jnp.zeros((8, 64), jnp.float32).at[:4].set(sage1_wr)
    y1, s1, q1 = _layer1_y(agg1, t1, inv, wl8, _v(sage1_bl), wr8)
    m1, r1 = _mr(s1, q1)
    h1a, h1b = _bn_relu_split(y1, m1, r1, _v(cbn1_w), _v(cbn1_b))

    a2a = _segsum(_seg32, h1a, srcp, dstl, z32)
    a2b = _segsum(_seg32, h1b, srcp, dstl, z32)
    y2, s2, q2 = _layer2_y(a2a, a2b, h1a, h1b, inv,
                           sage2_wl, _v(sage2_bl), sage2_wr)
    m2, r2 = _mr(s2, q2)
    h2, p3a, p3b = _bn_relu_proj2(y2, m2, r2, _v(cbn2_w), _v(cbn2_b),
                                  sage3_wl[:, :32], sage3_wl[:, 32:])

    P3a = _segsum(_seg32, p3a, srcp, dstl, z32)
    P3b = _segsum(_seg32, p3b, srcp, dstl, z32)
    y3, s3, q3 = _layer3_y(P3a, P3b, h2, inv, _v(sage3_bl), sage3_wr)
    m3, r3 = _mr(s3, q3)
    h3, p4 = _bn_relu_proj1(y3, m3, r3, _v(cbn3_w), _v(cbn3_b), sage4_wl)

    P4 = _segsum(_seg32, p4, srcp, dstl, z32)
    y4, s4, q4 = _layer4_y(P4, h3, inv, _v(sage4_bl), sage4_wr)
    m4, r4 = _mr(s4, q4)
    out = _head(y4, m4, r4, _v(cbn4_w), _v(cbn4_b), out_w, _v(out_b))
    return out[:, 0]
